# baseline (speedup 1.0000x reference)
import functools

import jax
import jax.numpy as jnp
from jax import lax
from jax.experimental import pallas as pl
from jax.experimental.pallas import tpu as pltpu
from jax.experimental.pallas import tpu_sc as plsc

B = 4096
S = 200
D = 64
C = 16
PAD = 208
HALF = PAD // 2
NC = 2
NS = 16
NW = NC * NS
BPW = B // NW


def _sc_pool(x2, table):
    mesh = plsc.VectorSubcoreMesh(
        core_axis_name="c", subcore_axis_name="s",
        num_cores=NC, num_subcores=NS)

    @functools.partial(
        pl.kernel,
        out_type=jax.ShapeDtypeStruct((B, D), jnp.float32),
        mesh=mesh,
        scratch_types=[
            pltpu.VMEM((2 * BPW, HALF), jnp.int32),
            pltpu.VMEM((PAD, D), jnp.float32),
            pltpu.VMEM((BPW, D), jnp.float32),
            pltpu.SemaphoreType.DMA,
        ],
        compiler_params=pltpu.CompilerParams(use_tc_tiling_on_sc=False),
    )
    def k(x_hbm, tab_hbm, out_hbm, idx_v, buf_v, out_v, sem):
        wid = lax.axis_index("s") * NC + lax.axis_index("c")
        pltpu.sync_copy(x_hbm.at[pl.ds(wid * 2 * BPW, 2 * BPW)], idx_v)

        def row(i, carry):
            c0 = pltpu.async_copy(
                tab_hbm.at[idx_v.at[2 * i]], buf_v.at[pl.ds(0, HALF)], sem)
            c1 = pltpu.async_copy(
                tab_hbm.at[idx_v.at[2 * i + 1]], buf_v.at[pl.ds(HALF, HALF)],
                sem)
            c0.wait()
            c1.wait()
            zero = jnp.zeros((16,), jnp.float32)

            def tok(s, accs):
                return tuple(accs[j] + buf_v[s, pl.ds(j * 16, 16)]
                             for j in range(4))

            a = lax.fori_loop(0, PAD, tok, (zero, zero, zero, zero),
                              unroll=4)
            for j in range(4):
                out_v[i, pl.ds(j * 16, 16)] = a[j]
            return carry

        lax.fori_loop(0, BPW, row, 0)
        pltpu.sync_copy(out_v, out_hbm.at[pl.ds(wid * BPW, BPW)])

    return k(x2, table)


def _linear(pooled, W, b2):
    def body(p_ref, w_ref, b_ref, o_ref):
        acc = lax.dot_general(
            p_ref[...], w_ref[...], (((1,), (1,)), ((), ())),
            preferred_element_type=jnp.float32)
        o_ref[...] = acc * (1.0 / S) + b_ref[...]

    return pl.pallas_call(
        body,
        out_shape=jax.ShapeDtypeStruct((B, C), jnp.float32),
    )(pooled, W, b2)


def kernel(x, table, W, b):
    x2 = jnp.pad(x, ((0, 0), (0, PAD - S))).reshape(2 * B, HALF)
    pooled = _sc_pool(x2, table)
    return _linear(pooled, W, b.reshape(1, C))


# speedup vs baseline: 1.8322x; 1.8322x over previous
import functools

import jax
import jax.numpy as jnp
from jax import lax
from jax.experimental import pallas as pl
from jax.experimental.pallas import tpu as pltpu
from jax.experimental.pallas import tpu_sc as plsc

B = 4096
S = 200
D = 64
C = 16
CH = 40
CPB = S // CH
NC = 2
NS = 16
NW = NC * NS
BPW = B // NW
NR = BPW * CPB


def _sc_pool(x5, table):
    mesh = plsc.VectorSubcoreMesh(
        core_axis_name="c", subcore_axis_name="s",
        num_cores=NC, num_subcores=NS)

    @functools.partial(
        pl.kernel,
        out_type=jax.ShapeDtypeStruct((B, D), jnp.float32),
        mesh=mesh,
        scratch_types=[
            pltpu.VMEM((NR, CH), jnp.int32),
            pltpu.VMEM((S, D), jnp.float32),
            pltpu.VMEM((S, D), jnp.float32),
            pltpu.VMEM((BPW, D), jnp.float32),
            pltpu.SemaphoreType.DMA,
            pltpu.SemaphoreType.DMA,
        ],
        compiler_params=pltpu.CompilerParams(use_tc_tiling_on_sc=False),
    )
    def k(x_hbm, tab_hbm, out_hbm, idx_v, buf0, buf1, out_v, sem0, sem1):
        wid = lax.axis_index("s") * NC + lax.axis_index("c")
        zero = jnp.zeros((16,), jnp.float32)
        pltpu.sync_copy(x_hbm.at[pl.ds(wid * NR, NR)], idx_v)

        def fire(i, buf, sem):
            for j in range(CPB):
                pltpu.async_copy(tab_hbm.at[idx_v.at[CPB * i + j]],
                                 buf.at[pl.ds(CH * j, CH)], sem)

        def drain(buf, sem):
            pltpu.make_async_copy(tab_hbm.at[pl.ds(0, S)], buf, sem).wait()

        def acc_row(i, buf):
            def tok(s, accs):
                return tuple(accs[j] + buf[s, pl.ds(j * 16, 16)]
                             for j in range(4))
            a = lax.fori_loop(0, S, tok, (zero, zero, zero, zero),
                              unroll=8)
            for j in range(4):
                out_v[i, pl.ds(j * 16, 16)] = a[j]

        fire(0, buf0, sem0)

        def pair(p, carry):
            i0 = 2 * p
            fire(i0 + 1, buf1, sem1)
            drain(buf0, sem0)
            acc_row(i0, buf0)

            @pl.when(i0 + 2 < BPW)
            def _():
                fire(i0 + 2, buf0, sem0)
            drain(buf1, sem1)
            acc_row(i0 + 1, buf1)
            return carry

        lax.fori_loop(0, BPW // 2, pair, 0)
        pltpu.sync_copy(out_v, out_hbm.at[pl.ds(wid * BPW, BPW)])

    return k(x5, table)


def _linear(pooled, W, b2):
    def body(p_ref, w_ref, b_ref, o_ref):
        acc = lax.dot_general(
            p_ref[...], w_ref[...], (((1,), (1,)), ((), ())),
            preferred_element_type=jnp.float32)
        o_ref[...] = acc * (1.0 / S) + b_ref[...]

    return pl.pallas_call(
        body,
        out_shape=jax.ShapeDtypeStruct((B, C), jnp.float32),
    )(pooled, W, b2)


def kernel(x, table, W, b):
    x5 = x.reshape(B * CPB, CH)
    pooled = _sc_pool(x5, table)
    return _linear(pooled, W, b.reshape(1, C))
